# baseline (speedup 1.0000x reference)
import functools

import jax
import jax.numpy as jnp
from jax import lax
from jax.experimental import pallas as pl
from jax.experimental.pallas import tpu as pltpu
from jax.experimental.pallas import tpu_sc as plsc

_NUM_EMB = 1000000
_DIM = 32
_BATCH = 16384
_HIST = 50

_NC = 2
_NS = 16
_NW = _NC * _NS
_BPW = _BATCH // _NW
_G = 128
_NG = _BPW // _G
_BT = _BATCH // _G
_DT = _DIM // 8

_mesh = plsc.VectorSubcoreMesh(core_axis_name="c", subcore_axis_name="s")


@functools.partial(
    pl.kernel,
    mesh=_mesh,
    out_type=jax.ShapeDtypeStruct((_HIST, _DT, _BT, 8, _G), jnp.float32),
    compiler_params=pltpu.CompilerParams(
        use_tc_tiling_on_sc=False, needs_layout_passes=False
    ),
    scratch_types=[
        pltpu.VMEM((_HIST, _NG, _G), jnp.int32),
        pltpu.VMEM((2, _BPW, _DIM), jnp.float32),
        pltpu.VMEM((2, _DT, _NG, 8, _G), jnp.float32),
        [pltpu.SemaphoreType.DMA] * 2,
        [pltpu.SemaphoreType.DMA] * 2,
    ],
)
def _embed_sc(idx_hbm, table_hbm, out_hbm, idx_v, rows_v, trows_v, gsems, osems):
    wid = lax.axis_index("s") * _NC + lax.axis_index("c")

    pltpu.sync_copy(idx_hbm.at[:, pl.ds(wid * _NG, _NG)], idx_v)

    lanes = lax.iota(jnp.int32, 16)

    def fire(h, b):
        for k in range(_NG):
            pltpu.async_copy(
                table_hbm.at[idx_v.at[h, k]],
                rows_v.at[b, pl.ds(k * _G, _G)],
                gsems[b],
            )

    def drain_g(b):
        pltpu.make_async_copy(table_hbm.at[pl.ds(0, _BPW)], rows_v.at[b], gsems[b]).wait()

    def transpose(b):
        @plsc.parallel_loop(0, _DIM, unroll=8)
        def step(d):
            for bt in range(_NG):
                for g in range(_G // 16):
                    rows = plsc.load_gather(
                        rows_v.at[b],
                        [bt * _G + g * 16 + lanes,
                         jnp.full((16,), d, jnp.int32)],
                    )
                    trows_v[b, d // 8, bt, d % 8, pl.ds(g * 16, 16)] = rows

    def fire_out(h, b):
        pltpu.async_copy(
            trows_v.at[b],
            out_hbm.at[h, :, pl.ds(wid * _NG, _NG)],
            osems[b],
        )

    def drain_o(b):
        pltpu.make_async_copy(trows_v.at[b], out_hbm.at[0, :, pl.ds(0, _NG)], osems[b]).wait()

    fire(0, 0)

    def pair(i, carry):
        h = 2 * i
        drain_g(0)
        fire(h + 1, 1)

        @pl.when(i > 0)
        def _():
            drain_o(0)

        transpose(0)
        fire_out(h, 0)

        drain_g(1)

        @pl.when(i < _HIST // 2 - 1)
        def _():
            fire(h + 2, 0)

        @pl.when(i > 0)
        def _():
            drain_o(1)

        transpose(1)
        fire_out(h + 1, 1)
        return carry

    lax.fori_loop(0, _HIST // 2, pair, 0)
    drain_o(0)
    drain_o(1)


def kernel(input_, weight):
    idx = input_.astype(jnp.int32).T.reshape(_HIST, _BT, _G)
    out5 = _embed_sc(idx, weight)
    return out5.transpose(2, 4, 0, 1, 3).reshape(_BATCH, _HIST, _DIM)


# speedup vs baseline: 1.3709x; 1.3709x over previous
import functools

import jax
import jax.numpy as jnp
from jax import lax
from jax.experimental import pallas as pl
from jax.experimental.pallas import tpu as pltpu
from jax.experimental.pallas import tpu_sc as plsc

_NUM_EMB = 1000000
_DIM = 32
_BATCH = 16384
_HIST = 50

_NC = 2
_NS = 16
_NW = _NC * _NS
_BPW = _BATCH // _NW
_G = 128
_NG = _BPW // _G
_BT = _BATCH // _G
_DT = _DIM // 8

_mesh = plsc.VectorSubcoreMesh(core_axis_name="c", subcore_axis_name="s")


@functools.partial(
    pl.kernel,
    mesh=_mesh,
    out_type=jax.ShapeDtypeStruct((_HIST, _DT, _BT, 8, _G), jnp.float32),
    compiler_params=pltpu.CompilerParams(
        use_tc_tiling_on_sc=False, needs_layout_passes=False
    ),
    scratch_types=[
        pltpu.VMEM((_HIST, _NG, _G), jnp.int32),
        pltpu.VMEM((2, _BPW, _DIM), jnp.float32),
        pltpu.VMEM((2, _DT, _NG, 8, _G), jnp.float32),
        [pltpu.SemaphoreType.DMA] * 2,
        [pltpu.SemaphoreType.DMA] * 2,
    ],
)
def _embed_sc(idx_hbm, table_hbm, out_hbm, idx_v, rows_v, trows_v, gsems, osems):
    wid = lax.axis_index("s") * _NC + lax.axis_index("c")

    pltpu.sync_copy(idx_hbm.at[:, pl.ds(wid * _NG, _NG)], idx_v)

    lanes = lax.iota(jnp.int32, 16)

    def fire(h, b):
        for k in range(_NG):
            pltpu.async_copy(
                table_hbm.at[idx_v.at[h, k]],
                rows_v.at[b, pl.ds(k * _G, _G)],
                gsems[b],
            )

    def drain_g(b):
        pltpu.make_async_copy(table_hbm.at[pl.ds(0, _BPW)], rows_v.at[b], gsems[b]).wait()

    rowsel = [(lanes + s) & 15 for s in range(16)]
    dcol = [lanes, lanes + 16]
    dt_c = [d >> 3 for d in dcol]
    ds_c = [d & 7 for d in dcol]

    def transpose(b):
        @plsc.parallel_loop(0, _BPW // 16, unroll=2)
        def step(bblk):
            b0 = bblk * 16
            btl_vec = jnp.full((16,), b0 // _G, jnp.int32)
            bl0 = b0 % _G
            for s in range(16):
                bsel = b0 + rowsel[s]
                blsel = bl0 + rowsel[s]
                for c in range(2):
                    vals = plsc.load_gather(rows_v.at[b], [bsel, dcol[c]])
                    plsc.store_scatter(
                        trows_v.at[b],
                        [dt_c[c], btl_vec, ds_c[c], blsel],
                        vals,
                    )

    def fire_out(h, b):
        pltpu.async_copy(
            trows_v.at[b],
            out_hbm.at[h, :, pl.ds(wid * _NG, _NG)],
            osems[b],
        )

    def drain_o(b):
        pltpu.make_async_copy(trows_v.at[b], out_hbm.at[0, :, pl.ds(0, _NG)], osems[b]).wait()

    fire(0, 0)

    def pair(i, carry):
        h = 2 * i
        drain_g(0)
        fire(h + 1, 1)

        @pl.when(i > 0)
        def _():
            drain_o(0)

        transpose(0)
        fire_out(h, 0)

        drain_g(1)

        @pl.when(i < _HIST // 2 - 1)
        def _():
            fire(h + 2, 0)

        @pl.when(i > 0)
        def _():
            drain_o(1)

        transpose(1)
        fire_out(h + 1, 1)
        return carry

    lax.fori_loop(0, _HIST // 2, pair, 0)
    drain_o(0)
    drain_o(1)


def kernel(input_, weight):
    idx = input_.astype(jnp.int32).T.reshape(_HIST, _BT, _G)
    out5 = _embed_sc(idx, weight)
    return out5.transpose(2, 4, 0, 1, 3).reshape(_BATCH, _HIST, _DIM)
